# Initial kernel scaffold; baseline (speedup 1.0000x reference)
import jax
import jax.numpy as jnp
from jax.experimental import pallas as pl


def kernel(embeds, labels):
    raise NotImplementedError("write your pallas kernel here")


# speedup vs baseline: 26.3283x; 26.3283x over previous
import jax
import jax.numpy as jnp
from jax import lax
from jax.experimental import pallas as pl
from jax.experimental.pallas import tpu as pltpu
from jax.experimental.pallas import tpu_sc as plsc

DELTA_V = 0.5
DELTA_D = 1.5
ALPHA = 1.0
BETA = 1.0
GAMMA = 0.001
NCLS = 16
EPS = 1e-12

FDIM = 32
NPIX = 512 * 512
NTILE = 16
PIX_PER_TILE = NPIX // NTILE
PCHUNK = 1024
NCHUNK = PIX_PER_TILE // PCHUNK
NGRP = PCHUNK // 16

TWORDS = (FDIM + 1) * NCLS * 16
TWORDS_PAD = 12288
CNT_OFF = FDIM * NCLS * 16

STAGE_OFF = 0
GLOB_OFF = NTILE * TWORDS_PAD
VAR_OFF = GLOB_OFF + TWORDS_PAD
SH_WORDS = VAR_OFF + 2 * NTILE * 16
SLICE = TWORDS_PAD // NTILE


def _sqrt16(x):
    i = plsc.bitcast(x, jnp.int32)
    i = jnp.int32(0x5F3759DF) - (i >> 1)
    r = plsc.bitcast(i, jnp.float32)
    for _ in range(3):
        r = r * (1.5 - 0.5 * x * r * r)
    return x * r


def _dl_body(emb_hbm, lab_hbm, out_hbm, sh, table, gtab, emb_buf, lab_buf,
             meansT, invc, presf, varbuf, vread, obuf):
    c = lax.axis_index("c")
    s = lax.axis_index("s")
    iota = lax.iota(jnp.int32, 16)
    zeros = jnp.zeros((16,), jnp.float32)
    ones = jnp.ones((16,), jnp.float32)
    zi = jnp.zeros((16,), jnp.int32)

    def zero_table(i, _):
        table[pl.ds(i * 16, 16)] = zeros
        return 0

    lax.fori_loop(0, TWORDS_PAD // 16, zero_table, 0)

    per_image = []
    for b_local in range(2):
        b = c * 2 + b_local
        pix0 = s * PIX_PER_TILE

        def chunk1(k, _):
            base = pix0 + k * PCHUNK
            pltpu.sync_copy(emb_hbm.at[b, :, pl.ds(base, PCHUNK)], emb_buf)
            pltpu.sync_copy(lab_hbm.at[b, pl.ds(base, PCHUNK)], lab_buf)

            def grp(g, _g):
                lab16 = lab_buf[pl.ds(g * 16, 16)]
                lbase = lab16 * 16 + iota
                plsc.addupdate_scatter(table, [lbase + CNT_OFF], ones)
                for f in range(FDIM):
                    v = emb_buf[f, pl.ds(g * 16, 16)]
                    plsc.addupdate_scatter(table, [lbase + f * (NCLS * 16)], v)
                return 0

            lax.fori_loop(0, NGRP, grp, 0)
            return 0

        lax.fori_loop(0, NCHUNK, chunk1, 0)

        pltpu.sync_copy(table, sh.at[pl.ds(s * TWORDS_PAD, TWORDS_PAD)])
        plsc.subcore_barrier()

        def red(u, _):
            pltpu.sync_copy(
                sh.at[pl.ds(u * TWORDS_PAD + s * SLICE, SLICE)],
                gtab.at[pl.ds(u * SLICE, SLICE)])
            return 0

        lax.fori_loop(0, NTILE, red, 0)

        def sumw(w, _):
            acc = gtab[pl.ds(w * 16, 16)]
            for u in range(1, NTILE):
                acc = acc + gtab[pl.ds(u * SLICE + w * 16, 16)]
            gtab[pl.ds(w * 16, 16)] = acc
            return 0

        lax.fori_loop(0, SLICE // 16, sumw, 0)
        pltpu.sync_copy(gtab.at[pl.ds(0, SLICE)],
                        sh.at[pl.ds(GLOB_OFF + s * SLICE, SLICE)])
        plsc.subcore_barrier()
        pltpu.sync_copy(sh.at[pl.ds(GLOB_OFF, TWORDS_PAD)], gtab)

        cacc = zeros
        for lane in range(16):
            cacc = cacc + plsc.load_gather(gtab,
                                           [iota * 16 + (CNT_OFF + lane)])
        counts = cacc
        present = counts > 0.0
        pres_f = jnp.where(present, 1.0, 0.0).astype(jnp.float32)
        safe = jnp.where(present, counts, 1.0)
        invc[...] = 1.0 / safe
        presf[...] = pres_f
        C_b = jnp.full((16,), jnp.sum(pres_f), jnp.float32)

        def meanrow(f, _):
            acc = zeros
            for lane in range(16):
                acc = acc + plsc.load_gather(
                    gtab, [iota * 16 + (f * (NCLS * 16) + lane)])
            meansT[pl.ds(f * 16, 16)] = acc / safe
            return 0

        lax.fori_loop(0, FDIM, meanrow, 0)

        def regf(f, acc):
            m = meansT[pl.ds(f * 16, 16)]
            return acc + m * m

        nrm2 = lax.fori_loop(0, FDIM, regf, zeros)
        norms = _sqrt16(nrm2 + EPS)
        reg_sum = jnp.full((16,), jnp.sum(jnp.where(present, norms, 0.0)),
                           jnp.float32)
        reg_b = jnp.where(C_b > 1.0, reg_sum, 0.0)

        def disrow(i, acc):
            def df(f, a):
                mi = plsc.load_gather(meansT, [zi + (f * 16 + i)])
                mrow = meansT[pl.ds(f * 16, 16)]
                d = mrow - mi
                return a + d * d

            d2 = lax.fori_loop(0, FDIM, df, zeros)
            dmat = _sqrt16(d2 + EPS)
            h = jnp.maximum(DELTA_D - dmat, 0.0)
            h = h * h
            pi = plsc.load_gather(presf, [zi + i])
            msk = jnp.where(iota > i, pres_f, 0.0) * pi
            return acc + h * msk

        pair_vec = lax.fori_loop(0, NCLS, disrow, zeros)
        pair_sum = jnp.full((16,), jnp.sum(pair_vec), jnp.float32)
        denom = jnp.maximum(C_b * (C_b - 1.0), 1.0)
        dis_b = jnp.where(C_b > 2.0, pair_sum / denom, 0.0)
        per_image.append((dis_b, reg_b, C_b))

        def chunk2(k, vacc):
            base = pix0 + k * PCHUNK
            pltpu.sync_copy(emb_hbm.at[b, :, pl.ds(base, PCHUNK)], emb_buf)
            pltpu.sync_copy(lab_hbm.at[b, pl.ds(base, PCHUNK)], lab_buf)

            def grp(g, va):
                lab16 = lab_buf[pl.ds(g * 16, 16)]
                acc = jnp.full((16,), EPS, jnp.float32)
                for f in range(FDIM):
                    v = emb_buf[f, pl.ds(g * 16, 16)]
                    m = plsc.load_gather(meansT, [lab16 + f * 16])
                    d = v - m
                    acc = acc + d * d
                dist = _sqrt16(acc)
                h = jnp.maximum(dist - DELTA_V, 0.0)
                ic = plsc.load_gather(invc, [lab16])
                return va + h * h * ic

            return lax.fori_loop(0, NGRP, grp, vacc)

        var_vec = lax.fori_loop(0, NCHUNK, chunk2, zeros)
        varbuf[...] = var_vec
        slot = VAR_OFF + (b_local * NTILE + s) * 16
        pltpu.sync_copy(varbuf, sh.at[pl.ds(slot, 16)])

        if b_local == 0:
            lax.fori_loop(0, TWORDS_PAD // 16, zero_table, 0)

    plsc.subcore_barrier()

    pltpu.sync_copy(sh.at[pl.ds(VAR_OFF, 2 * NTILE * 16)], vread)
    loss = zeros
    for b_local in range(2):
        dis_b, reg_b, C_b = per_image[b_local]

        def vrow(t, a):
            return a + vread[pl.ds((b_local * NTILE + t) * 16, 16)]

        vsum = lax.fori_loop(0, NTILE, vrow, zeros)
        var_sum = jnp.full((16,), jnp.sum(vsum), jnp.float32)
        var_b = jnp.where(C_b > 0.0, var_sum / jnp.maximum(C_b, 1.0), 0.0)
        loss = loss + ALPHA * var_b + BETA * dis_b + GAMMA * reg_b

    obuf[...] = jnp.where(iota == 0, loss, 0.0)

    @pl.when(s == 0)
    def _():
        pltpu.sync_copy(obuf, out_hbm.at[c])


def _make_call():
    mesh = plsc.VectorSubcoreMesh(core_axis_name="c", subcore_axis_name="s")
    return pl.kernel(
        _dl_body,
        out_type=jax.ShapeDtypeStruct((2, 16), jnp.float32),
        mesh=mesh,
        compiler_params=pltpu.CompilerParams(needs_layout_passes=False),
        scratch_types=[
            pltpu.VMEM_SHARED((SH_WORDS,), jnp.float32),
            pltpu.VMEM((TWORDS_PAD,), jnp.float32),
            pltpu.VMEM((TWORDS_PAD,), jnp.float32),
            pltpu.VMEM((FDIM, PCHUNK), jnp.float32),
            pltpu.VMEM((PCHUNK,), jnp.int32),
            pltpu.VMEM((NCLS * FDIM,), jnp.float32),
            pltpu.VMEM((16,), jnp.float32),
            pltpu.VMEM((16,), jnp.float32),
            pltpu.VMEM((16,), jnp.float32),
            pltpu.VMEM((2 * NTILE * 16,), jnp.float32),
            pltpu.VMEM((16,), jnp.float32),
        ],
    )


@jax.jit
def kernel(embeds, labels):
    B, F_, H, W = embeds.shape
    emb = embeds.reshape(B, F_, H * W)
    lab = labels.reshape(B, H * W)
    out = _make_call()(emb, lab)
    return jnp.sum(out)


# speedup vs baseline: 33.0076x; 1.2537x over previous
import jax
import jax.numpy as jnp
from jax import lax
from jax.experimental import pallas as pl
from jax.experimental.pallas import tpu as pltpu
from jax.experimental.pallas import tpu_sc as plsc

DELTA_V = 0.5
DELTA_D = 1.5
ALPHA = 1.0
BETA = 1.0
GAMMA = 0.001
NCLS = 16
EPS = 1e-12

FDIM = 32
NPIX = 512 * 512
NTILE = 16
PIX_PER_TILE = NPIX // NTILE
PCHUNK = 1024
NCHUNK = PIX_PER_TILE // PCHUNK
NGRP = PCHUNK // 16

TWORDS = (FDIM + 1) * NCLS * 16
TWORDS_PAD = 12288
CNT_OFF = FDIM * NCLS * 16

STAGE_OFF = 0
GLOB_OFF = NTILE * TWORDS_PAD
VAR_OFF = GLOB_OFF + TWORDS_PAD
SH_WORDS = VAR_OFF + 2 * NTILE * 16
SLICE = TWORDS_PAD // NTILE


def _sqrt16(x):
    i = plsc.bitcast(x, jnp.int32)
    i = jnp.int32(0x5F3759DF) - (i >> 1)
    r = plsc.bitcast(i, jnp.float32)
    for _ in range(3):
        r = r * (1.5 - 0.5 * x * r * r)
    return x * r


def _dl_body(emb_hbm, lab_hbm, out_hbm, sh, table, gtab, emb_buf, lab_buf,
             meansT, invc, presf, varbuf, vread, obuf,
             sem_e0, sem_e1, sem_l0, sem_l1):
    c = lax.axis_index("c")
    s = lax.axis_index("s")
    sem_e = (sem_e0, sem_e1)
    sem_l = (sem_l0, sem_l1)
    iota = lax.iota(jnp.int32, 16)
    zeros = jnp.zeros((16,), jnp.float32)
    ones = jnp.ones((16,), jnp.float32)
    zi = jnp.zeros((16,), jnp.int32)

    def zero_table(i, _):
        table[pl.ds(i * 16, 16)] = zeros
        return 0

    lax.fori_loop(0, TWORDS_PAD // 16, zero_table, 0)

    def stream_chunks(b, pix0, process, carry_init):
        def start(k, slot):
            base = pix0 + k * PCHUNK
            pltpu.async_copy(emb_hbm.at[b, :, pl.ds(base, PCHUNK)],
                             emb_buf.at[slot], sem_e[slot])
            pltpu.async_copy(lab_hbm.at[b, pl.ds(base, PCHUNK)],
                             lab_buf.at[slot], sem_l[slot])

        def wait(slot):
            pltpu.make_async_copy(emb_hbm.at[b, :, pl.ds(pix0, PCHUNK)],
                                  emb_buf.at[slot], sem_e[slot]).wait()
            pltpu.make_async_copy(lab_hbm.at[b, pl.ds(pix0, PCHUNK)],
                                  lab_buf.at[slot], sem_l[slot]).wait()

        start(0, 0)

        def body(kk, car):
            k0 = kk * 2
            start(k0 + 1, 1)
            wait(0)
            car = process(0, car)
            start(lax.rem(k0 + 2, NCHUNK), 0)
            wait(1)
            car = process(1, car)
            return car

        car = lax.fori_loop(0, NCHUNK // 2, body, carry_init)
        wait(0)
        return car

    per_image = []
    for b_local in range(2):
        b = c * 2 + b_local
        pix0 = s * PIX_PER_TILE

        def proc1(slot, car):
            def grp(g, _g):
                lab16 = lab_buf[slot, pl.ds(g * 16, 16)]
                lbase = lab16 * 16 + iota
                plsc.addupdate_scatter(table, [lbase + CNT_OFF], ones)
                for f in range(FDIM):
                    v = emb_buf[slot, f, pl.ds(g * 16, 16)]
                    plsc.addupdate_scatter(table, [lbase + f * (NCLS * 16)], v)
                return 0

            lax.fori_loop(0, NGRP, grp, 0)
            return car

        stream_chunks(b, pix0, proc1, 0)

        pltpu.sync_copy(table, sh.at[pl.ds(s * TWORDS_PAD, TWORDS_PAD)])
        plsc.subcore_barrier()

        def red(u, _):
            pltpu.sync_copy(
                sh.at[pl.ds(u * TWORDS_PAD + s * SLICE, SLICE)],
                gtab.at[pl.ds(u * SLICE, SLICE)])
            return 0

        lax.fori_loop(0, NTILE, red, 0)

        def sumw(w, _):
            acc = gtab[pl.ds(w * 16, 16)]
            for u in range(1, NTILE):
                acc = acc + gtab[pl.ds(u * SLICE + w * 16, 16)]
            gtab[pl.ds(w * 16, 16)] = acc
            return 0

        lax.fori_loop(0, SLICE // 16, sumw, 0)
        pltpu.sync_copy(gtab.at[pl.ds(0, SLICE)],
                        sh.at[pl.ds(GLOB_OFF + s * SLICE, SLICE)])
        plsc.subcore_barrier()
        pltpu.sync_copy(sh.at[pl.ds(GLOB_OFF, TWORDS_PAD)], gtab)

        cacc = zeros
        for lane in range(16):
            cacc = cacc + plsc.load_gather(gtab,
                                           [iota * 16 + (CNT_OFF + lane)])
        counts = cacc
        present = counts > 0.0
        pres_f = jnp.where(present, 1.0, 0.0).astype(jnp.float32)
        safe = jnp.where(present, counts, 1.0)
        invc[...] = 1.0 / safe
        presf[...] = pres_f
        C_b = jnp.full((16,), jnp.sum(pres_f), jnp.float32)

        def meanrow(f, _):
            acc = zeros
            for lane in range(16):
                acc = acc + plsc.load_gather(
                    gtab, [iota * 16 + (f * (NCLS * 16) + lane)])
            meansT[pl.ds(f * 16, 16)] = acc / safe
            return 0

        lax.fori_loop(0, FDIM, meanrow, 0)

        def regf(f, acc):
            m = meansT[pl.ds(f * 16, 16)]
            return acc + m * m

        nrm2 = lax.fori_loop(0, FDIM, regf, zeros)
        norms = _sqrt16(nrm2 + EPS)
        reg_sum = jnp.full((16,), jnp.sum(jnp.where(present, norms, 0.0)),
                           jnp.float32)
        reg_b = jnp.where(C_b > 1.0, reg_sum, 0.0)

        def disrow(i, acc):
            def df(f, a):
                mi = plsc.load_gather(meansT, [zi + (f * 16 + i)])
                mrow = meansT[pl.ds(f * 16, 16)]
                d = mrow - mi
                return a + d * d

            d2 = lax.fori_loop(0, FDIM, df, zeros)
            dmat = _sqrt16(d2 + EPS)
            h = jnp.maximum(DELTA_D - dmat, 0.0)
            h = h * h
            pi = plsc.load_gather(presf, [zi + i])
            msk = jnp.where(iota > i, pres_f, 0.0) * pi
            return acc + h * msk

        pair_vec = lax.fori_loop(0, NCLS, disrow, zeros)
        pair_sum = jnp.full((16,), jnp.sum(pair_vec), jnp.float32)
        denom = jnp.maximum(C_b * (C_b - 1.0), 1.0)
        dis_b = jnp.where(C_b > 2.0, pair_sum / denom, 0.0)
        per_image.append((dis_b, reg_b, C_b))

        def proc2(slot, vacc):
            def grp(g, va):
                lab16 = lab_buf[slot, pl.ds(g * 16, 16)]
                acc = jnp.full((16,), EPS, jnp.float32)
                for f in range(FDIM):
                    v = emb_buf[slot, f, pl.ds(g * 16, 16)]
                    m = plsc.load_gather(meansT, [lab16 + f * 16])
                    d = v - m
                    acc = acc + d * d
                dist = _sqrt16(acc)
                h = jnp.maximum(dist - DELTA_V, 0.0)
                ic = plsc.load_gather(invc, [lab16])
                return va + h * h * ic

            return lax.fori_loop(0, NGRP, grp, vacc)

        var_vec = stream_chunks(b, pix0, proc2, zeros)
        varbuf[...] = var_vec
        slot = VAR_OFF + (b_local * NTILE + s) * 16
        pltpu.sync_copy(varbuf, sh.at[pl.ds(slot, 16)])

        if b_local == 0:
            lax.fori_loop(0, TWORDS_PAD // 16, zero_table, 0)

    plsc.subcore_barrier()

    pltpu.sync_copy(sh.at[pl.ds(VAR_OFF, 2 * NTILE * 16)], vread)
    loss = zeros
    for b_local in range(2):
        dis_b, reg_b, C_b = per_image[b_local]

        def vrow(t, a):
            return a + vread[pl.ds((b_local * NTILE + t) * 16, 16)]

        vsum = lax.fori_loop(0, NTILE, vrow, zeros)
        var_sum = jnp.full((16,), jnp.sum(vsum), jnp.float32)
        var_b = jnp.where(C_b > 0.0, var_sum / jnp.maximum(C_b, 1.0), 0.0)
        loss = loss + ALPHA * var_b + BETA * dis_b + GAMMA * reg_b

    obuf[...] = jnp.where(iota == 0, loss, 0.0)

    @pl.when(s == 0)
    def _():
        pltpu.sync_copy(obuf, out_hbm.at[c])


def _make_call():
    mesh = plsc.VectorSubcoreMesh(core_axis_name="c", subcore_axis_name="s")
    return pl.kernel(
        _dl_body,
        out_type=jax.ShapeDtypeStruct((2, 16), jnp.float32),
        mesh=mesh,
        compiler_params=pltpu.CompilerParams(needs_layout_passes=False),
        scratch_types=[
            pltpu.VMEM_SHARED((SH_WORDS,), jnp.float32),
            pltpu.VMEM((TWORDS_PAD,), jnp.float32),
            pltpu.VMEM((TWORDS_PAD,), jnp.float32),
            pltpu.VMEM((2, FDIM, PCHUNK), jnp.float32),
            pltpu.VMEM((2, PCHUNK), jnp.int32),
            pltpu.VMEM((NCLS * FDIM,), jnp.float32),
            pltpu.VMEM((16,), jnp.float32),
            pltpu.VMEM((16,), jnp.float32),
            pltpu.VMEM((16,), jnp.float32),
            pltpu.VMEM((2 * NTILE * 16,), jnp.float32),
            pltpu.VMEM((16,), jnp.float32),
            pltpu.SemaphoreType.DMA,
            pltpu.SemaphoreType.DMA,
            pltpu.SemaphoreType.DMA,
            pltpu.SemaphoreType.DMA,
        ],
    )


@jax.jit
def kernel(embeds, labels):
    B, F_, H, W = embeds.shape
    emb = embeds.reshape(B, F_, H * W)
    lab = labels.reshape(B, H * W)
    out = _make_call()(emb, lab)
    return jnp.sum(out)


# speedup vs baseline: 39.8740x; 1.2080x over previous
import jax
import jax.numpy as jnp
from jax import lax
from jax.experimental import pallas as pl
from jax.experimental.pallas import tpu as pltpu
from jax.experimental.pallas import tpu_sc as plsc

DELTA_V = 0.5
DELTA_D = 1.5
ALPHA = 1.0
BETA = 1.0
GAMMA = 0.001
NCLS = 16
EPS = 1e-12

FDIM = 32
NPIX = 512 * 512
NTILE = 16
PIX_PER_TILE = NPIX // NTILE
PCHUNK = 1024
NCHUNK = PIX_PER_TILE // PCHUNK
NGRP = PCHUNK // 16

TWORDS = (FDIM + 1) * NCLS * 16
TWORDS_PAD = 12288
CNT_OFF = FDIM * NCLS * 16

STAGE_OFF = 0
GLOB_OFF = NTILE * TWORDS_PAD
VAR_OFF = GLOB_OFF + TWORDS_PAD
SH_WORDS = VAR_OFF + 2 * NTILE * 16
SLICE = TWORDS_PAD // NTILE


def _take16(vec, idx):
    dn = lax.GatherDimensionNumbers(offset_dims=(), collapsed_slice_dims=(0,),
                                    start_index_map=(0,))
    return lax.gather(vec, idx[:, None], dn, (1,),
                      mode=lax.GatherScatterMode.PROMISE_IN_BOUNDS)


def _sqrt16(x):
    i = plsc.bitcast(x, jnp.int32)
    i = jnp.int32(0x5F3759DF) - (i >> 1)
    r = plsc.bitcast(i, jnp.float32)
    for _ in range(3):
        r = r * (1.5 - 0.5 * x * r * r)
    return x * r


def _dl_body(emb_hbm, lab_hbm, out_hbm, sh, table, gtab, emb_buf, lab_buf,
             meansT, invc, presf, varbuf, vread, obuf,
             sem_e0, sem_e1, sem_l0, sem_l1):
    c = lax.axis_index("c")
    s = lax.axis_index("s")
    sem_e = (sem_e0, sem_e1)
    sem_l = (sem_l0, sem_l1)
    iota = lax.iota(jnp.int32, 16)
    zeros = jnp.zeros((16,), jnp.float32)
    ones = jnp.ones((16,), jnp.float32)
    zi = jnp.zeros((16,), jnp.int32)

    def zero_table(i, _):
        table[pl.ds(i * 16, 16)] = zeros
        return 0

    lax.fori_loop(0, TWORDS_PAD // 16, zero_table, 0)

    def stream_chunks(b, pix0, process, carry_init):
        def start(k, slot):
            base = pix0 + k * PCHUNK
            pltpu.async_copy(emb_hbm.at[b, :, pl.ds(base, PCHUNK)],
                             emb_buf.at[slot], sem_e[slot])
            pltpu.async_copy(lab_hbm.at[b, pl.ds(base, PCHUNK)],
                             lab_buf.at[slot], sem_l[slot])

        def wait(slot):
            pltpu.make_async_copy(emb_hbm.at[b, :, pl.ds(pix0, PCHUNK)],
                                  emb_buf.at[slot], sem_e[slot]).wait()
            pltpu.make_async_copy(lab_hbm.at[b, pl.ds(pix0, PCHUNK)],
                                  lab_buf.at[slot], sem_l[slot]).wait()

        start(0, 0)

        def body(kk, car):
            k0 = kk * 2
            start(k0 + 1, 1)
            wait(0)
            car = process(0, car)
            start(lax.rem(k0 + 2, NCHUNK), 0)
            wait(1)
            car = process(1, car)
            return car

        car = lax.fori_loop(0, NCHUNK // 2, body, carry_init)
        wait(0)
        return car

    per_image = []
    for b_local in range(2):
        b = c * 2 + b_local
        pix0 = s * PIX_PER_TILE

        def proc1(slot, car):
            @plsc.parallel_loop(0, NGRP, unroll=2)
            def grp(g):
                lab16 = lab_buf[slot, pl.ds(g * 16, 16)]
                lbase = lab16 * 16 + iota
                plsc.addupdate_scatter(table, [lbase + CNT_OFF], ones)
                for f in range(FDIM):
                    v = emb_buf[slot, f, pl.ds(g * 16, 16)]
                    plsc.addupdate_scatter(table, [lbase + f * (NCLS * 16)], v)

            return car

        stream_chunks(b, pix0, proc1, 0)

        pltpu.sync_copy(table, sh.at[pl.ds(s * TWORDS_PAD, TWORDS_PAD)])
        plsc.subcore_barrier()

        def red(u, _):
            pltpu.sync_copy(
                sh.at[pl.ds(u * TWORDS_PAD + s * SLICE, SLICE)],
                gtab.at[pl.ds(u * SLICE, SLICE)])
            return 0

        lax.fori_loop(0, NTILE, red, 0)

        def sumw(w, _):
            acc = gtab[pl.ds(w * 16, 16)]
            for u in range(1, NTILE):
                acc = acc + gtab[pl.ds(u * SLICE + w * 16, 16)]
            gtab[pl.ds(w * 16, 16)] = acc
            return 0

        lax.fori_loop(0, SLICE // 16, sumw, 0)
        pltpu.sync_copy(gtab.at[pl.ds(0, SLICE)],
                        sh.at[pl.ds(GLOB_OFF + s * SLICE, SLICE)])
        plsc.subcore_barrier()
        pltpu.sync_copy(sh.at[pl.ds(GLOB_OFF, TWORDS_PAD)], gtab)

        cacc = zeros
        for lane in range(16):
            cacc = cacc + plsc.load_gather(gtab,
                                           [iota * 16 + (CNT_OFF + lane)])
        counts = cacc
        present = counts > 0.0
        pres_f = jnp.where(present, 1.0, 0.0).astype(jnp.float32)
        safe = jnp.where(present, counts, 1.0)
        invc[...] = 1.0 / safe
        presf[...] = pres_f
        C_b = jnp.full((16,), jnp.sum(pres_f), jnp.float32)

        def meanrow(f, _):
            acc = zeros
            for lane in range(16):
                acc = acc + plsc.load_gather(
                    gtab, [iota * 16 + (f * (NCLS * 16) + lane)])
            meansT[pl.ds(f * 16, 16)] = acc / safe
            return 0

        lax.fori_loop(0, FDIM, meanrow, 0)

        def regf(f, acc):
            m = meansT[pl.ds(f * 16, 16)]
            return acc + m * m

        nrm2 = lax.fori_loop(0, FDIM, regf, zeros)
        norms = _sqrt16(nrm2 + EPS)
        reg_sum = jnp.full((16,), jnp.sum(jnp.where(present, norms, 0.0)),
                           jnp.float32)
        reg_b = jnp.where(C_b > 1.0, reg_sum, 0.0)

        def disrow(i, acc):
            def df(f, a):
                mi = plsc.load_gather(meansT, [zi + (f * 16 + i)])
                mrow = meansT[pl.ds(f * 16, 16)]
                d = mrow - mi
                return a + d * d

            d2 = lax.fori_loop(0, FDIM, df, zeros)
            dmat = _sqrt16(d2 + EPS)
            h = jnp.maximum(DELTA_D - dmat, 0.0)
            h = h * h
            pi = plsc.load_gather(presf, [zi + i])
            msk = jnp.where(iota > i, pres_f, 0.0) * pi
            return acc + h * msk

        pair_vec = lax.fori_loop(0, NCLS, disrow, zeros)
        pair_sum = jnp.full((16,), jnp.sum(pair_vec), jnp.float32)
        denom = jnp.maximum(C_b * (C_b - 1.0), 1.0)
        dis_b = jnp.where(C_b > 2.0, pair_sum / denom, 0.0)
        per_image.append((dis_b, reg_b, C_b))

        mrows = [meansT[pl.ds(f * 16, 16)] for f in range(FDIM)]
        inv_vec = 1.0 / safe

        def proc2(slot, vacc):
            def grp(g, va):
                lab16 = lab_buf[slot, pl.ds(g * 16, 16)]
                accs = [jnp.full((16,), EPS, jnp.float32), zeros, zeros, zeros]
                for f in range(FDIM):
                    v = emb_buf[slot, f, pl.ds(g * 16, 16)]
                    m = _take16(mrows[f], lab16)
                    d = v - m
                    accs[f % 4] = accs[f % 4] + d * d
                acc = (accs[0] + accs[1]) + (accs[2] + accs[3])
                dist = _sqrt16(acc)
                h = jnp.maximum(dist - DELTA_V, 0.0)
                ic = _take16(inv_vec, lab16)
                return va + h * h * ic

            return plsc.parallel_loop(0, NGRP, unroll=2, carry=vacc)(grp)

        var_vec = stream_chunks(b, pix0, proc2, zeros)
        varbuf[...] = var_vec
        slot = VAR_OFF + (b_local * NTILE + s) * 16
        pltpu.sync_copy(varbuf, sh.at[pl.ds(slot, 16)])

        if b_local == 0:
            lax.fori_loop(0, TWORDS_PAD // 16, zero_table, 0)

    plsc.subcore_barrier()

    pltpu.sync_copy(sh.at[pl.ds(VAR_OFF, 2 * NTILE * 16)], vread)
    loss = zeros
    for b_local in range(2):
        dis_b, reg_b, C_b = per_image[b_local]

        def vrow(t, a):
            return a + vread[pl.ds((b_local * NTILE + t) * 16, 16)]

        vsum = lax.fori_loop(0, NTILE, vrow, zeros)
        var_sum = jnp.full((16,), jnp.sum(vsum), jnp.float32)
        var_b = jnp.where(C_b > 0.0, var_sum / jnp.maximum(C_b, 1.0), 0.0)
        loss = loss + ALPHA * var_b + BETA * dis_b + GAMMA * reg_b

    obuf[...] = jnp.where(iota == 0, loss, 0.0)

    @pl.when(s == 0)
    def _():
        pltpu.sync_copy(obuf, out_hbm.at[c])


def _make_call():
    mesh = plsc.VectorSubcoreMesh(core_axis_name="c", subcore_axis_name="s")
    return pl.kernel(
        _dl_body,
        out_type=jax.ShapeDtypeStruct((2, 16), jnp.float32),
        mesh=mesh,
        compiler_params=pltpu.CompilerParams(needs_layout_passes=False),
        scratch_types=[
            pltpu.VMEM_SHARED((SH_WORDS,), jnp.float32),
            pltpu.VMEM((TWORDS_PAD,), jnp.float32),
            pltpu.VMEM((TWORDS_PAD,), jnp.float32),
            pltpu.VMEM((2, FDIM, PCHUNK), jnp.float32),
            pltpu.VMEM((2, PCHUNK), jnp.int32),
            pltpu.VMEM((NCLS * FDIM,), jnp.float32),
            pltpu.VMEM((16,), jnp.float32),
            pltpu.VMEM((16,), jnp.float32),
            pltpu.VMEM((16,), jnp.float32),
            pltpu.VMEM((2 * NTILE * 16,), jnp.float32),
            pltpu.VMEM((16,), jnp.float32),
            pltpu.SemaphoreType.DMA,
            pltpu.SemaphoreType.DMA,
            pltpu.SemaphoreType.DMA,
            pltpu.SemaphoreType.DMA,
        ],
    )


@jax.jit
def kernel(embeds, labels):
    B, F_, H, W = embeds.shape
    emb = embeds.reshape(B, F_, H * W)
    lab = labels.reshape(B, H * W)
    out = _make_call()(emb, lab)
    return jnp.sum(out)


# speedup vs baseline: 58.2883x; 1.4618x over previous
import jax
import jax.numpy as jnp
from jax import lax
from jax.experimental import pallas as pl
from jax.experimental.pallas import tpu as pltpu
from jax.experimental.pallas import tpu_sc as plsc

DELTA_V = 0.5
DELTA_D = 1.5
ALPHA = 1.0
BETA = 1.0
GAMMA = 0.001
NCLS = 16
EPS = 1e-12

FDIM = 32
HDIM = 512
WDIM = 512
NTILE = 16
ROWS_PER_TILE = HDIM // NTILE
CROWS = 2
NCHUNK = ROWS_PER_TILE // CROWS
NGRP_ROW = WDIM // 16

TWORDS = (FDIM + 1) * NCLS * 16
TWORDS_PAD = 12288
CNT_OFF = FDIM * NCLS * 16

STAGE_OFF = 0
GLOB_OFF = NTILE * TWORDS_PAD
VAR_OFF = GLOB_OFF + TWORDS_PAD
SH_WORDS = VAR_OFF + 2 * NTILE * 16
SLICE = TWORDS_PAD // NTILE


def _take16(vec, idx):
    dn = lax.GatherDimensionNumbers(offset_dims=(), collapsed_slice_dims=(0,),
                                    start_index_map=(0,))
    return lax.gather(vec, idx[:, None], dn, (1,),
                      mode=lax.GatherScatterMode.PROMISE_IN_BOUNDS)


def _sqrt16(x):
    i = plsc.bitcast(x, jnp.int32)
    i = jnp.int32(0x5F3759DF) - (i >> 1)
    r = plsc.bitcast(i, jnp.float32)
    for _ in range(3):
        r = r * (1.5 - 0.5 * x * r * r)
    return x * r


def _dl_body(emb_hbm, lab_hbm, out_hbm, sh, table, gtab, emb_buf, lab_buf,
             meansT, invc, presf, varbuf, vread, obuf,
             sem_e0, sem_e1, sem_l0, sem_l1):
    c = lax.axis_index("c")
    s = lax.axis_index("s")
    sem_e = (sem_e0, sem_e1)
    sem_l = (sem_l0, sem_l1)
    iota = lax.iota(jnp.int32, 16)
    zeros = jnp.zeros((16,), jnp.float32)
    ones = jnp.ones((16,), jnp.float32)
    zi = jnp.zeros((16,), jnp.int32)

    def zero_table(i, _):
        table[pl.ds(i * 16, 16)] = zeros
        return 0

    lax.fori_loop(0, TWORDS_PAD // 16, zero_table, 0)

    def stream_chunks(b, row0, process, carry_init):
        def start(k, slot):
            base = row0 + k * CROWS
            pltpu.async_copy(emb_hbm.at[b, :, pl.ds(base, CROWS), :],
                             emb_buf.at[slot], sem_e[slot])
            pltpu.async_copy(lab_hbm.at[b, pl.ds(base, CROWS), :],
                             lab_buf.at[slot], sem_l[slot])

        def wait(slot):
            pltpu.make_async_copy(emb_hbm.at[b, :, pl.ds(row0, CROWS), :],
                                  emb_buf.at[slot], sem_e[slot]).wait()
            pltpu.make_async_copy(lab_hbm.at[b, pl.ds(row0, CROWS), :],
                                  lab_buf.at[slot], sem_l[slot]).wait()

        start(0, 0)

        def body(kk, car):
            k0 = kk * 2
            start(k0 + 1, 1)
            wait(0)
            car = process(0, car)
            start(lax.rem(k0 + 2, NCHUNK), 0)
            wait(1)
            car = process(1, car)
            return car

        car = lax.fori_loop(0, NCHUNK // 2, body, carry_init)
        wait(0)
        return car

    per_image = []
    for b_local in range(2):
        b = c * 2 + b_local
        row0 = s * ROWS_PER_TILE

        def proc1(slot, car):
            @plsc.parallel_loop(0, CROWS * NGRP_ROW, unroll=2)
            def grp(g):
                r = g >> 5
                cc = (g & (NGRP_ROW - 1)) * 16
                lab16 = lab_buf[slot, r, pl.ds(cc, 16)]
                lbase = lab16 * 16 + iota
                plsc.addupdate_scatter(table, [lbase + CNT_OFF], ones)
                for f in range(FDIM):
                    v = emb_buf[slot, f, r, pl.ds(cc, 16)]
                    plsc.addupdate_scatter(table,
                                           [lbase + f * (NCLS * 16)], v)

            return car

        stream_chunks(b, row0, proc1, 0)

        pltpu.sync_copy(table, sh.at[pl.ds(s * TWORDS_PAD, TWORDS_PAD)])
        plsc.subcore_barrier()

        def red(u, _):
            pltpu.sync_copy(
                sh.at[pl.ds(u * TWORDS_PAD + s * SLICE, SLICE)],
                gtab.at[pl.ds(u * SLICE, SLICE)])
            return 0

        lax.fori_loop(0, NTILE, red, 0)

        def sumw(w, _):
            acc = gtab[pl.ds(w * 16, 16)]
            for u in range(1, NTILE):
                acc = acc + gtab[pl.ds(u * SLICE + w * 16, 16)]
            gtab[pl.ds(w * 16, 16)] = acc
            return 0

        lax.fori_loop(0, SLICE // 16, sumw, 0)
        pltpu.sync_copy(gtab.at[pl.ds(0, SLICE)],
                        sh.at[pl.ds(GLOB_OFF + s * SLICE, SLICE)])
        plsc.subcore_barrier()
        pltpu.sync_copy(sh.at[pl.ds(GLOB_OFF, TWORDS_PAD)], gtab)

        cacc = zeros
        for lane in range(16):
            cacc = cacc + plsc.load_gather(gtab,
                                           [iota * 16 + (CNT_OFF + lane)])
        counts = cacc
        present = counts > 0.0
        pres_f = jnp.where(present, 1.0, 0.0).astype(jnp.float32)
        safe = jnp.where(present, counts, 1.0)
        invc[...] = 1.0 / safe
        presf[...] = pres_f
        C_b = jnp.full((16,), jnp.sum(pres_f), jnp.float32)

        def meanrow(f, _):
            acc = zeros
            for lane in range(16):
                acc = acc + plsc.load_gather(
                    gtab, [iota * 16 + (f * (NCLS * 16) + lane)])
            meansT[pl.ds(f * 16, 16)] = acc / safe
            return 0

        lax.fori_loop(0, FDIM, meanrow, 0)

        def regf(f, acc):
            m = meansT[pl.ds(f * 16, 16)]
            return acc + m * m

        nrm2 = lax.fori_loop(0, FDIM, regf, zeros)
        norms = _sqrt16(nrm2 + EPS)
        reg_sum = jnp.full((16,), jnp.sum(jnp.where(present, norms, 0.0)),
                           jnp.float32)
        reg_b = jnp.where(C_b > 1.0, reg_sum, 0.0)

        def disrow(i, acc):
            def df(f, a):
                mi = plsc.load_gather(meansT, [zi + (f * 16 + i)])
                mrow = meansT[pl.ds(f * 16, 16)]
                d = mrow - mi
                return a + d * d

            d2 = lax.fori_loop(0, FDIM, df, zeros)
            dmat = _sqrt16(d2 + EPS)
            h = jnp.maximum(DELTA_D - dmat, 0.0)
            h = h * h
            pi = plsc.load_gather(presf, [zi + i])
            msk = jnp.where(iota > i, pres_f, 0.0) * pi
            return acc + h * msk

        pair_vec = lax.fori_loop(0, NCLS, disrow, zeros)
        pair_sum = jnp.full((16,), jnp.sum(pair_vec), jnp.float32)
        denom = jnp.maximum(C_b * (C_b - 1.0), 1.0)
        dis_b = jnp.where(C_b > 2.0, pair_sum / denom, 0.0)
        per_image.append((dis_b, reg_b, C_b))

        mrows = [meansT[pl.ds(f * 16, 16)] for f in range(FDIM)]
        inv_vec = 1.0 / safe

        def proc2(slot, vacc):
            def grp(g, va):
                r = g >> 5
                cc = (g & (NGRP_ROW - 1)) * 16
                lab16 = lab_buf[slot, r, pl.ds(cc, 16)]
                accs = [jnp.full((16,), EPS, jnp.float32),
                        zeros, zeros, zeros]
                for f in range(FDIM):
                    v = emb_buf[slot, f, r, pl.ds(cc, 16)]
                    m = _take16(mrows[f], lab16)
                    d = v - m
                    accs[f % 4] = accs[f % 4] + d * d
                acc = (accs[0] + accs[1]) + (accs[2] + accs[3])
                dist = _sqrt16(acc)
                h = jnp.maximum(dist - DELTA_V, 0.0)
                ic = _take16(inv_vec, lab16)
                return va + h * h * ic

            return plsc.parallel_loop(0, CROWS * NGRP_ROW, unroll=2,
                                      carry=vacc)(grp)

        var_vec = stream_chunks(b, row0, proc2, zeros)
        varbuf[...] = var_vec
        slot = VAR_OFF + (b_local * NTILE + s) * 16
        pltpu.sync_copy(varbuf, sh.at[pl.ds(slot, 16)])

        if b_local == 0:
            lax.fori_loop(0, TWORDS_PAD // 16, zero_table, 0)

    plsc.subcore_barrier()

    pltpu.sync_copy(sh.at[pl.ds(VAR_OFF, 2 * NTILE * 16)], vread)
    loss = zeros
    for b_local in range(2):
        dis_b, reg_b, C_b = per_image[b_local]

        def vrow(t, a):
            return a + vread[pl.ds((b_local * NTILE + t) * 16, 16)]

        vsum = lax.fori_loop(0, NTILE, vrow, zeros)
        var_sum = jnp.full((16,), jnp.sum(vsum), jnp.float32)
        var_b = jnp.where(C_b > 0.0, var_sum / jnp.maximum(C_b, 1.0), 0.0)
        loss = loss + ALPHA * var_b + BETA * dis_b + GAMMA * reg_b

    obuf[...] = jnp.where(iota == 0, loss, 0.0)

    @pl.when(s == 0)
    def _():
        pltpu.sync_copy(obuf, out_hbm.at[c])


def _make_call():
    mesh = plsc.VectorSubcoreMesh(core_axis_name="c", subcore_axis_name="s")
    return pl.kernel(
        _dl_body,
        out_type=jax.ShapeDtypeStruct((2, 16), jnp.float32),
        mesh=mesh,
        compiler_params=pltpu.CompilerParams(needs_layout_passes=False),
        scratch_types=[
            pltpu.VMEM_SHARED((SH_WORDS,), jnp.float32),
            pltpu.VMEM((TWORDS_PAD,), jnp.float32),
            pltpu.VMEM((TWORDS_PAD,), jnp.float32),
            pltpu.VMEM((2, FDIM, CROWS, WDIM), jnp.float32),
            pltpu.VMEM((2, CROWS, WDIM), jnp.int32),
            pltpu.VMEM((NCLS * FDIM,), jnp.float32),
            pltpu.VMEM((16,), jnp.float32),
            pltpu.VMEM((16,), jnp.float32),
            pltpu.VMEM((16,), jnp.float32),
            pltpu.VMEM((2 * NTILE * 16,), jnp.float32),
            pltpu.VMEM((16,), jnp.float32),
            pltpu.SemaphoreType.DMA,
            pltpu.SemaphoreType.DMA,
            pltpu.SemaphoreType.DMA,
            pltpu.SemaphoreType.DMA,
        ],
    )


@jax.jit
def kernel(embeds, labels):
    B, F_, H, W = embeds.shape
    lab = labels.reshape(B, H, W)
    out = _make_call()(embeds, lab)
    return jnp.sum(out)


# speedup vs baseline: 61.0553x; 1.0475x over previous
import jax
import jax.numpy as jnp
from jax import lax
from jax.experimental import pallas as pl
from jax.experimental.pallas import tpu as pltpu
from jax.experimental.pallas import tpu_sc as plsc

DELTA_V = 0.5
DELTA_D = 1.5
ALPHA = 1.0
BETA = 1.0
GAMMA = 0.001
NCLS = 16
EPS = 1e-12

FDIM = 32
HDIM = 512
WDIM = 512
NTILE = 16
ROWS_PER_TILE = HDIM // NTILE
CROWS = 2
NCHUNK = ROWS_PER_TILE // CROWS
NGRP_ROW = WDIM // 16

TWORDS = (FDIM + 1) * NCLS * 16
CNT_OFF = FDIM * NCLS * 16

STAGE_OFF = 0
GLOB_OFF = NTILE * TWORDS
VAR_OFF = GLOB_OFF + TWORDS
SH_WORDS = VAR_OFF + 2 * NTILE * 16
SLICE = TWORDS // NTILE


def _take16(vec, idx):
    dn = lax.GatherDimensionNumbers(offset_dims=(), collapsed_slice_dims=(0,),
                                    start_index_map=(0,))
    return lax.gather(vec, idx[:, None], dn, (1,),
                      mode=lax.GatherScatterMode.PROMISE_IN_BOUNDS)


def _sqrt16(x):
    i = plsc.bitcast(x, jnp.int32)
    i = jnp.int32(0x5F3759DF) - (i >> 1)
    r = plsc.bitcast(i, jnp.float32)
    for _ in range(3):
        r = r * (1.5 - 0.5 * x * r * r)
    return x * r


def _dl_body(emb_hbm, lab_hbm, out_hbm, sh, table, gtab, emb_buf, lab_buf,
             meansT, invc, presf, varbuf, vread, obuf,
             sem_e0, sem_e1, sem_l0, sem_l1):
    c = lax.axis_index("c")
    s = lax.axis_index("s")
    sem_e = (sem_e0, sem_e1)
    sem_l = (sem_l0, sem_l1)
    iota = lax.iota(jnp.int32, 16)
    zeros = jnp.zeros((16,), jnp.float32)
    ones = jnp.ones((16,), jnp.float32)
    zi = jnp.zeros((16,), jnp.int32)

    def zero_table(i, _):
        table[pl.ds(i * 16, 16)] = zeros
        return 0

    lax.fori_loop(0, TWORDS // 16, zero_table, 0)

    def start_at(bb, base, slot):
        pltpu.async_copy(emb_hbm.at[bb, :, pl.ds(base, CROWS), :],
                         emb_buf.at[slot], sem_e[slot])
        pltpu.async_copy(lab_hbm.at[bb, pl.ds(base, CROWS), :],
                         lab_buf.at[slot], sem_l[slot])

    def wait_slot(slot):
        pltpu.make_async_copy(emb_hbm.at[0, :, pl.ds(0, CROWS), :],
                              emb_buf.at[slot], sem_e[slot]).wait()
        pltpu.make_async_copy(lab_hbm.at[0, pl.ds(0, CROWS), :],
                              lab_buf.at[slot], sem_l[slot]).wait()

    def stream_chunks(b, row0, process, carry_init, primed, tail_b,
                      tail_row0):
        if not primed:
            start_at(b, row0, 0)

        def body(kk, car):
            k0 = kk * 2
            start_at(b, row0 + (k0 + 1) * CROWS, 1)
            wait_slot(0)
            car = process(0, car)
            nk = k0 + 2
            last = nk >= NCHUNK
            bb = jnp.where(last, tail_b, b)
            base = jnp.where(last, tail_row0, row0 + nk * CROWS)
            start_at(bb, base, 0)
            wait_slot(1)
            car = process(1, car)
            return car

        return lax.fori_loop(0, NCHUNK // 2, body, carry_init)

    per_image = []
    for b_local in range(2):
        b = c * 2 + b_local
        row0 = s * ROWS_PER_TILE

        def proc1(slot, car):
            @plsc.parallel_loop(0, CROWS * NGRP_ROW, unroll=2)
            def grp(g):
                r = g >> 5
                cc = (g & (NGRP_ROW - 1)) * 16
                lab16 = lab_buf[slot, r, pl.ds(cc, 16)]
                lbase = lab16 * 16 + iota
                plsc.addupdate_scatter(table, [lbase + CNT_OFF], ones)
                for f in range(FDIM):
                    v = emb_buf[slot, f, r, pl.ds(cc, 16)]
                    plsc.addupdate_scatter(table,
                                           [lbase + f * (NCLS * 16)], v)

            return car

        stream_chunks(b, row0, proc1, 0, primed=(b_local > 0),
                      tail_b=b, tail_row0=row0)

        pltpu.sync_copy(table, sh.at[pl.ds(s * TWORDS, TWORDS)])
        plsc.subcore_barrier()

        def red(u, _):
            pltpu.sync_copy(
                sh.at[pl.ds(u * TWORDS + s * SLICE, SLICE)],
                gtab.at[pl.ds(u * SLICE, SLICE)])
            return 0

        lax.fori_loop(0, NTILE, red, 0)

        def sumw(w, _):
            acc = gtab[pl.ds(w * 16, 16)]
            for u in range(1, NTILE):
                acc = acc + gtab[pl.ds(u * SLICE + w * 16, 16)]
            gtab[pl.ds(w * 16, 16)] = acc
            return 0

        lax.fori_loop(0, SLICE // 16, sumw, 0)
        pltpu.sync_copy(gtab.at[pl.ds(0, SLICE)],
                        sh.at[pl.ds(GLOB_OFF + s * SLICE, SLICE)])
        plsc.subcore_barrier()
        pltpu.sync_copy(sh.at[pl.ds(GLOB_OFF, TWORDS)], gtab)

        cacc = zeros
        for lane in range(16):
            cacc = cacc + plsc.load_gather(gtab,
                                           [iota * 16 + (CNT_OFF + lane)])
        counts = cacc
        present = counts > 0.0
        pres_f = jnp.where(present, 1.0, 0.0).astype(jnp.float32)
        safe = jnp.where(present, counts, 1.0)
        invc[...] = 1.0 / safe
        presf[...] = pres_f
        C_b = jnp.full((16,), jnp.sum(pres_f), jnp.float32)

        def meanrow(f, _):
            acc = zeros
            for lane in range(16):
                acc = acc + plsc.load_gather(
                    gtab, [iota * 16 + (f * (NCLS * 16) + lane)])
            meansT[pl.ds(f * 16, 16)] = acc / safe
            return 0

        lax.fori_loop(0, FDIM, meanrow, 0)

        def regf(f, acc):
            m = meansT[pl.ds(f * 16, 16)]
            return acc + m * m

        nrm2 = lax.fori_loop(0, FDIM, regf, zeros)
        norms = _sqrt16(nrm2 + EPS)
        reg_sum = jnp.full((16,), jnp.sum(jnp.where(present, norms, 0.0)),
                           jnp.float32)
        reg_b = jnp.where(C_b > 1.0, reg_sum, 0.0)

        def disrow(i, acc):
            def df(f, a):
                mi = plsc.load_gather(meansT, [zi + (f * 16 + i)])
                mrow = meansT[pl.ds(f * 16, 16)]
                d = mrow - mi
                return a + d * d

            d2 = lax.fori_loop(0, FDIM, df, zeros)
            dmat = _sqrt16(d2 + EPS)
            h = jnp.maximum(DELTA_D - dmat, 0.0)
            h = h * h
            pi = plsc.load_gather(presf, [zi + i])
            msk = jnp.where(iota > i, pres_f, 0.0) * pi
            return acc + h * msk

        pair_vec = lax.fori_loop(0, NCLS, disrow, zeros)
        pair_sum = jnp.full((16,), jnp.sum(pair_vec), jnp.float32)
        denom = jnp.maximum(C_b * (C_b - 1.0), 1.0)
        dis_b = jnp.where(C_b > 2.0, pair_sum / denom, 0.0)
        per_image.append((dis_b, reg_b, C_b))

        mrows = [meansT[pl.ds(f * 16, 16)] for f in range(FDIM)]
        inv_vec = 1.0 / safe

        def proc2(slot, vacc):
            def grp(g, va):
                r = g >> 5
                cc = (g & (NGRP_ROW - 1)) * 16
                lab16 = lab_buf[slot, r, pl.ds(cc, 16)]
                accs = [jnp.full((16,), EPS, jnp.float32),
                        zeros, zeros, zeros]
                for f in range(FDIM):
                    v = emb_buf[slot, f, r, pl.ds(cc, 16)]
                    m = _take16(mrows[f], lab16)
                    d = v - m
                    accs[f % 4] = accs[f % 4] + d * d
                acc = (accs[0] + accs[1]) + (accs[2] + accs[3])
                dist = _sqrt16(acc)
                h = jnp.maximum(dist - DELTA_V, 0.0)
                ic = _take16(inv_vec, lab16)
                return va + h * h * ic

            return plsc.parallel_loop(0, CROWS * NGRP_ROW, unroll=2,
                                      carry=vacc)(grp)

        var_vec = stream_chunks(b, row0, proc2, zeros, primed=True,
                                tail_b=jnp.minimum(b + 1, 2 * c + 1),
                                tail_row0=row0)
        varbuf[...] = var_vec
        slot = VAR_OFF + (b_local * NTILE + s) * 16
        pltpu.sync_copy(varbuf, sh.at[pl.ds(slot, 16)])

        if b_local == 0:
            lax.fori_loop(0, TWORDS // 16, zero_table, 0)

    wait_slot(0)
    plsc.subcore_barrier()

    pltpu.sync_copy(sh.at[pl.ds(VAR_OFF, 2 * NTILE * 16)], vread)
    loss = zeros
    for b_local in range(2):
        dis_b, reg_b, C_b = per_image[b_local]

        def vrow(t, a):
            return a + vread[pl.ds((b_local * NTILE + t) * 16, 16)]

        vsum = lax.fori_loop(0, NTILE, vrow, zeros)
        var_sum = jnp.full((16,), jnp.sum(vsum), jnp.float32)
        var_b = jnp.where(C_b > 0.0, var_sum / jnp.maximum(C_b, 1.0), 0.0)
        loss = loss + ALPHA * var_b + BETA * dis_b + GAMMA * reg_b

    obuf[...] = jnp.where(iota == 0, loss, 0.0)

    @pl.when(s == 0)
    def _():
        pltpu.sync_copy(obuf, out_hbm.at[c])


def _make_call():
    mesh = plsc.VectorSubcoreMesh(core_axis_name="c", subcore_axis_name="s")
    return pl.kernel(
        _dl_body,
        out_type=jax.ShapeDtypeStruct((2, 16), jnp.float32),
        mesh=mesh,
        compiler_params=pltpu.CompilerParams(needs_layout_passes=False),
        scratch_types=[
            pltpu.VMEM_SHARED((SH_WORDS,), jnp.float32),
            pltpu.VMEM((TWORDS,), jnp.float32),
            pltpu.VMEM((TWORDS,), jnp.float32),
            pltpu.VMEM((2, FDIM, CROWS, WDIM), jnp.float32),
            pltpu.VMEM((2, CROWS, WDIM), jnp.int32),
            pltpu.VMEM((NCLS * FDIM,), jnp.float32),
            pltpu.VMEM((16,), jnp.float32),
            pltpu.VMEM((16,), jnp.float32),
            pltpu.VMEM((16,), jnp.float32),
            pltpu.VMEM((2 * NTILE * 16,), jnp.float32),
            pltpu.VMEM((16,), jnp.float32),
            pltpu.SemaphoreType.DMA,
            pltpu.SemaphoreType.DMA,
            pltpu.SemaphoreType.DMA,
            pltpu.SemaphoreType.DMA,
        ],
    )


@jax.jit
def kernel(embeds, labels):
    B, F_, H, W = embeds.shape
    lab = labels.reshape(B, H, W)
    out = _make_call()(embeds, lab)
    return jnp.sum(out)
